# baseline (speedup 1.0000x reference)
import functools

import jax
import jax.numpy as jnp
from jax import lax
from jax.experimental import pallas as pl
from jax.experimental.pallas import tpu as pltpu
from jax.experimental.pallas import tpu_sc as plsc

_L = 16
_NW = 32
_P = 96


def _warp_body(B, H, W, C,
               w2_hbm, feat_hbm, mask_hbm, out_hbm,
               wxv, wyv, mv,
               idx0, wt0, buf0, idx1, wt1, buf1,
               ob0, sem0, sem1, semo0):
    HW = H * W
    N = B * HW
    npw = N // _NW
    nchunks = npw // _P
    rows_pw = npw // W
    cpr = W // _P

    cid = lax.axis_index("c")
    sid = lax.axis_index("s")
    wid = sid * 2 + cid
    wbase = wid * npw
    b = wid // 16
    bimg = b * HW
    row0 = wid * rows_pw - b * H
    grow0 = wid * rows_pw

    feat2 = feat_hbm

    def prep(k, idxs, wts):
        q = k >> 2
        r = k - q * cpr
        row = row0 + q
        grow = grow0 + q
        colbase = r * _P
        pltpu.sync_copy(w2_hbm.at[2 * grow, pl.ds(colbase, _P)], wxv)
        pltpu.sync_copy(w2_hbm.at[2 * grow + 1, pl.ds(colbase, _P)], wyv)
        pltpu.sync_copy(mask_hbm.at[row, pl.ds(colbase, _P)], mv)

        def sub(t, _):
            off = t * _L
            col = colbase + off + lax.iota(jnp.int32, _L)
            jf = col.astype(jnp.float32)
            iff = jnp.full((_L,), row, jnp.int32).astype(jnp.float32)
            wx = wxv[pl.ds(off, _L)]
            wy = wyv[pl.ds(off, _L)]
            tgx = jf + wx
            tgy = iff + wy
            xs = 2.0 * tgx / float(W - 1) - 1.0
            ys = 2.0 * tgy / float(H - 1) - 1.0
            x = 0.5 * (xs + 1.0) * float(W)
            y = 0.5 * (ys + 1.0) * float(H)
            tx = x.astype(jnp.int32)
            ty = y.astype(jnp.int32)
            x0 = jnp.where(tx.astype(jnp.float32) > x, tx - 1, tx)
            y0 = jnp.where(ty.astype(jnp.float32) > y, ty - 1, ty)
            x0c = jnp.minimum(jnp.maximum(x0, 0), W - 1)
            x1c = jnp.minimum(jnp.maximum(x0 + 1, 0), W - 1)
            y0c = jnp.minimum(jnp.maximum(y0, 0), H - 1)
            y1c = jnp.minimum(jnp.maximum(y0 + 1, 0), H - 1)
            ry0 = bimg + y0c * W
            ry1 = bimg + y1c * W
            idxs[0, pl.ds(off, _L)] = ry0 + x0c
            idxs[1, pl.ds(off, _L)] = ry1 + x0c
            idxs[2, pl.ds(off, _L)] = ry0 + x1c
            idxs[3, pl.ds(off, _L)] = ry1 + x1c
            x0f = x0c.astype(jnp.float32)
            x1f = x1c.astype(jnp.float32)
            y0f = y0c.astype(jnp.float32)
            y1f = y1c.astype(jnp.float32)
            mk = mv[pl.ds(off, _L)]
            wts[0, pl.ds(off, _L)] = (x1f - x) * (y1f - y) * mk
            wts[1, pl.ds(off, _L)] = (x1f - x) * (y - y0f) * mk
            wts[2, pl.ds(off, _L)] = (x - x0f) * (y1f - y) * mk
            wts[3, pl.ds(off, _L)] = (x - x0f) * (y - y0f) * mk
            return 0

        lax.fori_loop(0, _P // _L, sub, 0)

    def fire(idxs, bufs, sem):
        for c4 in range(4):
            pltpu.make_async_copy(
                feat2.at[idxs.at[c4]], bufs.at[c4], sem).start()

    def wait(idxs, bufs, sem):
        for c4 in range(4):
            pltpu.make_async_copy(
                feat2.at[idxs.at[c4]], bufs.at[c4], sem).wait()

    def combine(wts, bufs, colbase):
        civs = [j * _L + lax.iota(jnp.int32, _L) for j in range(C // _L)]

        def px(p, _):
            pi = jnp.full((_L,), p, jnp.int32)
            pcol = jnp.full((_L,), colbase + p, jnp.int32)
            wa = plsc.load_gather(wts.at[0], [pi])
            wb = plsc.load_gather(wts.at[1], [pi])
            wc = plsc.load_gather(wts.at[2], [pi])
            wd = plsc.load_gather(wts.at[3], [pi])
            for j in range(C // _L):
                off = j * _L
                va = bufs[0, p, pl.ds(off, _L)]
                vb = bufs[1, p, pl.ds(off, _L)]
                vc = bufs[2, p, pl.ds(off, _L)]
                vd = bufs[3, p, pl.ds(off, _L)]
                val = (wa * va + wb * vb) + (wc * vc + wd * vd)
                plsc.store_scatter(ob0, [civs[j], pcol], val)
            return 0

        lax.fori_loop(0, _P, px, 0)

    def out_row(q):
        return out_hbm.at[pl.ds((grow0 + q) * C, C), :]

    def store(q):
        pltpu.make_async_copy(ob0, out_row(q), semo0).start()

    def drain(q):
        pltpu.make_async_copy(ob0, out_row(q), semo0).wait()

    prep(0, idx0, wt0)
    fire(idx0, buf0, sem0)

    slots = ((idx0, wt0, buf0, sem0), (idx1, wt1, buf1, sem1))

    def loop(q, _):
        k0 = 4 * q
        for rr in range(4):
            k = k0 + rr
            idxs, wts, bufs, sem = slots[rr % 2]
            nidxs, nwts, nbufs, nsem = slots[(rr + 1) % 2]
            if rr < 3:
                prep(k + 1, nidxs, nwts)
                fire(nidxs, nbufs, nsem)
            else:
                @pl.when(k + 1 < nchunks)
                def _():
                    prep(k + 1, nidxs, nwts)
                    fire(nidxs, nbufs, nsem)
            wait(idxs, bufs, sem)
            if rr == 0:
                @pl.when(q > 0)
                def _():
                    drain(q - 1)
            combine(wts, bufs, rr * _P)
        store(q)
        return 0

    lax.fori_loop(0, nchunks // 4, loop, 0)
    drain(nchunks // 4 - 1)


def kernel(w, feature, view_gp_mask):
    B, H, W, C = feature.shape
    N = B * H * W
    wx = w[:, :, :, 0].reshape(B * H, W)
    wy = w[:, :, :, 1].reshape(B * H, W)
    w2 = jnp.stack([wx, wy], axis=1).reshape(B * H * 2, W)
    featflat = feature.reshape(N, C)

    mesh = plsc.VectorSubcoreMesh(core_axis_name="c", subcore_axis_name="s")
    body = functools.partial(_warp_body, B, H, W, C)
    out = pl.kernel(
        body,
        out_type=jax.ShapeDtypeStruct((B * H * C, W), jnp.float32),
        mesh=mesh,
        compiler_params=pltpu.CompilerParams(
            needs_layout_passes=False, use_tc_tiling_on_sc=False),
        scratch_types=[
            pltpu.VMEM((_P,), jnp.float32),
            pltpu.VMEM((_P,), jnp.float32),
            pltpu.VMEM((_P,), jnp.float32),
            pltpu.VMEM((4, _P), jnp.int32),
            pltpu.VMEM((4, _P), jnp.float32),
            pltpu.VMEM((4, _P, C), jnp.float32),
            pltpu.VMEM((4, _P), jnp.int32),
            pltpu.VMEM((4, _P), jnp.float32),
            pltpu.VMEM((4, _P, C), jnp.float32),
            pltpu.VMEM((C, W), jnp.float32),
            pltpu.SemaphoreType.DMA,
            pltpu.SemaphoreType.DMA,
            pltpu.SemaphoreType.DMA,
        ],
    )(w2, featflat, view_gp_mask)

    return out.reshape(B, H, C, W).transpose(0, 1, 3, 2)


# speedup vs baseline: 1.3203x; 1.3203x over previous
import functools

import jax
import jax.numpy as jnp
from jax import lax
from jax.experimental import pallas as pl
from jax.experimental.pallas import tpu as pltpu
from jax.experimental.pallas import tpu_sc as plsc

_L = 16
_NW = 32
_P = 96


def _warp_body(B, H, W, C,
               w2_hbm, feat_hbm, mask_hbm, out_hbm,
               wxv, wyv, mv,
               idx0, wt0, buf0, idx1, wt1, buf1,
               ob0, sem0, sem1, semo0):
    HW = H * W
    N = B * HW
    npw = N // _NW
    nchunks = npw // _P
    rows_pw = npw // W
    cpr = W // _P

    cid = lax.axis_index("c")
    sid = lax.axis_index("s")
    wid = sid * 2 + cid
    wbase = wid * npw
    b = wid // 16
    bimg = b * HW
    row0 = wid * rows_pw - b * H
    grow0 = wid * rows_pw

    feat2 = feat_hbm

    def prep(k, idxs, wts):
        q = k >> 2
        r = k - q * cpr
        row = row0 + q
        grow = grow0 + q
        colbase = r * _P
        pltpu.sync_copy(w2_hbm.at[2 * grow, pl.ds(colbase, _P)], wxv)
        pltpu.sync_copy(w2_hbm.at[2 * grow + 1, pl.ds(colbase, _P)], wyv)
        pltpu.sync_copy(mask_hbm.at[row, pl.ds(colbase, _P)], mv)

        def sub(t, _):
            off = t * _L
            col = colbase + off + lax.iota(jnp.int32, _L)
            jf = col.astype(jnp.float32)
            iff = jnp.full((_L,), row, jnp.int32).astype(jnp.float32)
            wx = wxv[pl.ds(off, _L)]
            wy = wyv[pl.ds(off, _L)]
            tgx = jf + wx
            tgy = iff + wy
            xs = 2.0 * tgx / float(W - 1) - 1.0
            ys = 2.0 * tgy / float(H - 1) - 1.0
            x = 0.5 * (xs + 1.0) * float(W)
            y = 0.5 * (ys + 1.0) * float(H)
            tx = x.astype(jnp.int32)
            ty = y.astype(jnp.int32)
            x0 = jnp.where(tx.astype(jnp.float32) > x, tx - 1, tx)
            y0 = jnp.where(ty.astype(jnp.float32) > y, ty - 1, ty)
            x0c = jnp.minimum(jnp.maximum(x0, 0), W - 1)
            x1c = jnp.minimum(jnp.maximum(x0 + 1, 0), W - 1)
            y0c = jnp.minimum(jnp.maximum(y0, 0), H - 1)
            y1c = jnp.minimum(jnp.maximum(y0 + 1, 0), H - 1)
            ry0 = bimg + y0c * W
            ry1 = bimg + y1c * W
            idxs[0, pl.ds(off, _L)] = ry0 + x0c
            idxs[1, pl.ds(off, _L)] = ry1 + x0c
            idxs[2, pl.ds(off, _L)] = ry0 + x1c
            idxs[3, pl.ds(off, _L)] = ry1 + x1c
            x0f = x0c.astype(jnp.float32)
            x1f = x1c.astype(jnp.float32)
            y0f = y0c.astype(jnp.float32)
            y1f = y1c.astype(jnp.float32)
            mk = mv[pl.ds(off, _L)]
            wts[0, pl.ds(off, _L)] = (x1f - x) * (y1f - y) * mk
            wts[1, pl.ds(off, _L)] = (x1f - x) * (y - y0f) * mk
            wts[2, pl.ds(off, _L)] = (x - x0f) * (y1f - y) * mk
            wts[3, pl.ds(off, _L)] = (x - x0f) * (y - y0f) * mk
            return 0

        lax.fori_loop(0, _P // _L, sub, 0)

    def fire(idxs, bufs, sem):
        for c4 in range(4):
            pltpu.make_async_copy(
                feat2.at[idxs.at[c4]], bufs.at[c4], sem).start()

    def wait(idxs, bufs, sem):
        for c4 in range(4):
            pltpu.make_async_copy(
                feat2.at[idxs.at[c4]], bufs.at[c4], sem).wait()

    def combine(wts, bufs, colbase):
        civs = [j * _L + lax.iota(jnp.int32, _L) for j in range(C // _L)]

        def px(p, _):
            pi = jnp.full((_L,), p, jnp.int32)
            pcol = jnp.full((_L,), colbase + p, jnp.int32)
            wa = plsc.load_gather(wts.at[0], [pi])
            wb = plsc.load_gather(wts.at[1], [pi])
            wc = plsc.load_gather(wts.at[2], [pi])
            wd = plsc.load_gather(wts.at[3], [pi])
            for j in range(C // _L):
                off = j * _L
                va = bufs[0, p, pl.ds(off, _L)]
                vb = bufs[1, p, pl.ds(off, _L)]
                vc = bufs[2, p, pl.ds(off, _L)]
                vd = bufs[3, p, pl.ds(off, _L)]
                val = (wa * va + wb * vb) + (wc * vc + wd * vd)
                plsc.store_scatter(ob0, [civs[j], pcol], val)
            return 0

        lax.fori_loop(0, _P, px, 0)

    def out_row(q):
        return out_hbm.at[pl.ds((grow0 + q) * C, C), :]

    def store(q):
        pltpu.make_async_copy(ob0.at[:, pl.ds(0, W)], out_row(q), semo0).start()

    def drain(q):
        pltpu.make_async_copy(ob0.at[:, pl.ds(0, W)], out_row(q), semo0).wait()

    prep(0, idx0, wt0)
    fire(idx0, buf0, sem0)

    slots = ((idx0, wt0, buf0, sem0), (idx1, wt1, buf1, sem1))

    def loop(q, _):
        k0 = 4 * q
        for rr in range(4):
            k = k0 + rr
            idxs, wts, bufs, sem = slots[rr % 2]
            nidxs, nwts, nbufs, nsem = slots[(rr + 1) % 2]
            if rr < 3:
                prep(k + 1, nidxs, nwts)
                fire(nidxs, nbufs, nsem)
            else:
                @pl.when(k + 1 < nchunks)
                def _():
                    prep(k + 1, nidxs, nwts)
                    fire(nidxs, nbufs, nsem)
            wait(idxs, bufs, sem)
            if rr == 0:
                @pl.when(q > 0)
                def _():
                    drain(q - 1)
            combine(wts, bufs, rr * _P)
        store(q)
        return 0

    lax.fori_loop(0, nchunks // 4, loop, 0)
    drain(nchunks // 4 - 1)


def kernel(w, feature, view_gp_mask):
    B, H, W, C = feature.shape
    N = B * H * W
    wx = w[:, :, :, 0].reshape(B * H, W)
    wy = w[:, :, :, 1].reshape(B * H, W)
    w2 = jnp.stack([wx, wy], axis=1).reshape(B * H * 2, W)
    featflat = feature.reshape(N, C)

    mesh = plsc.VectorSubcoreMesh(core_axis_name="c", subcore_axis_name="s")
    body = functools.partial(_warp_body, B, H, W, C)
    out = pl.kernel(
        body,
        out_type=jax.ShapeDtypeStruct((B * H * C, W), jnp.float32),
        mesh=mesh,
        compiler_params=pltpu.CompilerParams(
            needs_layout_passes=False, use_tc_tiling_on_sc=False),
        scratch_types=[
            pltpu.VMEM((_P,), jnp.float32),
            pltpu.VMEM((_P,), jnp.float32),
            pltpu.VMEM((_P,), jnp.float32),
            pltpu.VMEM((4, _P), jnp.int32),
            pltpu.VMEM((4, _P), jnp.float32),
            pltpu.VMEM((4, _P, C), jnp.float32),
            pltpu.VMEM((4, _P), jnp.int32),
            pltpu.VMEM((4, _P), jnp.float32),
            pltpu.VMEM((4, _P, C), jnp.float32),
            pltpu.VMEM((C, W + 1), jnp.float32),
            pltpu.SemaphoreType.DMA,
            pltpu.SemaphoreType.DMA,
            pltpu.SemaphoreType.DMA,
        ],
    )(w2, featflat, view_gp_mask)

    return out.reshape(B, H, C, W).transpose(0, 1, 3, 2)


# speedup vs baseline: 1.7736x; 1.3433x over previous
import functools

import jax
import jax.numpy as jnp
from jax import lax
from jax.experimental import pallas as pl
from jax.experimental.pallas import tpu as pltpu
from jax.experimental.pallas import tpu_sc as plsc

_L = 16
_NW = 32
_P = 128


def _warp_body(B, H, W, C,
               wx_hbm, wy_hbm, feat_hbm, mask_hbm, out_hbm,
               wxv, wyv, mv,
               idx0, wt0, buf0, idx1, wt1, buf1,
               ob0, ob1, sem0, sem1, semo0, semo1):
    HW = H * W
    N = B * HW
    npw = N // _NW
    nchunks = npw // _P
    rows_pw = npw // W
    cpr = W // _P

    cid = lax.axis_index("c")
    sid = lax.axis_index("s")
    wid = sid * 2 + cid
    wbase = wid * npw
    b = wid // 16
    bimg = b * HW
    row0 = wid * rows_pw - b * H

    def prep(k, idxs, wts):
        base = wbase + k * _P
        q = (k * 21846) >> 16
        r = k - q * cpr
        row = row0 + q
        colbase = r * _P
        lpix = row * W + colbase
        pltpu.sync_copy(wx_hbm.at[pl.ds(base, _P)], wxv)
        pltpu.sync_copy(wy_hbm.at[pl.ds(base, _P)], wyv)
        pltpu.sync_copy(mask_hbm.at[pl.ds(lpix, _P)], mv)

        def sub(t, _):
            off = t * _L
            col = colbase + off + lax.iota(jnp.int32, _L)
            jf = col.astype(jnp.float32)
            iff = jnp.full((_L,), row, jnp.int32).astype(jnp.float32)
            wx = wxv[pl.ds(off, _L)]
            wy = wyv[pl.ds(off, _L)]
            tgx = jf + wx
            tgy = iff + wy
            xs = 2.0 * tgx / float(W - 1) - 1.0
            ys = 2.0 * tgy / float(H - 1) - 1.0
            x = 0.5 * (xs + 1.0) * float(W)
            y = 0.5 * (ys + 1.0) * float(H)
            tx = x.astype(jnp.int32)
            ty = y.astype(jnp.int32)
            x0 = jnp.where(tx.astype(jnp.float32) > x, tx - 1, tx)
            y0 = jnp.where(ty.astype(jnp.float32) > y, ty - 1, ty)
            x0c = jnp.minimum(jnp.maximum(x0, 0), W - 1)
            x1c = jnp.minimum(jnp.maximum(x0 + 1, 0), W - 1)
            y0c = jnp.minimum(jnp.maximum(y0, 0), H - 1)
            y1c = jnp.minimum(jnp.maximum(y0 + 1, 0), H - 1)
            ry0 = bimg + y0c * W
            ry1 = bimg + y1c * W
            idxs[0, pl.ds(off, _L)] = ry0 + x0c
            idxs[1, pl.ds(off, _L)] = ry1 + x0c
            idxs[2, pl.ds(off, _L)] = ry0 + x1c
            idxs[3, pl.ds(off, _L)] = ry1 + x1c
            x0f = x0c.astype(jnp.float32)
            x1f = x1c.astype(jnp.float32)
            y0f = y0c.astype(jnp.float32)
            y1f = y1c.astype(jnp.float32)
            mk = mv[pl.ds(off, _L)]
            wts[0, pl.ds(off, _L)] = (x1f - x) * (y1f - y) * mk
            wts[1, pl.ds(off, _L)] = (x1f - x) * (y - y0f) * mk
            wts[2, pl.ds(off, _L)] = (x - x0f) * (y1f - y) * mk
            wts[3, pl.ds(off, _L)] = (x - x0f) * (y - y0f) * mk
            return 0

        lax.fori_loop(0, _P // _L, sub, 0)

    def fire(idxs, bufs, sem):
        for c4 in range(4):
            pltpu.make_async_copy(
                feat_hbm.at[idxs.at[c4]], bufs.at[c4], sem).start()

    def wait(idxs, bufs, sem):
        for c4 in range(4):
            pltpu.make_async_copy(
                feat_hbm.at[idxs.at[c4]], bufs.at[c4], sem).wait()

    def combine(wts, bufs, obuf):
        def px(p, _):
            pi = jnp.full((_L,), p, jnp.int32)
            wa = plsc.load_gather(wts.at[0], [pi])
            wb = plsc.load_gather(wts.at[1], [pi])
            wc = plsc.load_gather(wts.at[2], [pi])
            wd = plsc.load_gather(wts.at[3], [pi])
            for cc in range(C // _L):
                off = cc * _L
                va = bufs[0, p, pl.ds(off, _L)]
                vb = bufs[1, p, pl.ds(off, _L)]
                vc = bufs[2, p, pl.ds(off, _L)]
                vd = bufs[3, p, pl.ds(off, _L)]
                obuf[p, pl.ds(off, _L)] = (wa * va + wb * vb) + (wc * vc + wd * vd)
            return 0

        lax.fori_loop(0, _P, px, 0)

    def store(k, obuf, semo):
        base = wbase + k * _P
        pltpu.make_async_copy(obuf, out_hbm.at[pl.ds(base, _P)], semo).start()

    def drain(k, obuf, semo):
        base = wbase + k * _P
        pltpu.make_async_copy(obuf, out_hbm.at[pl.ds(base, _P)], semo).wait()

    prep(0, idx0, wt0)
    fire(idx0, buf0, sem0)

    def loop(kk, _):
        k0 = 2 * kk
        k1 = 2 * kk + 1

        prep(k1, idx1, wt1)
        fire(idx1, buf1, sem1)
        wait(idx0, buf0, sem0)

        @pl.when(kk > 0)
        def _():
            drain(k0 - 2, ob0, semo0)

        combine(wt0, buf0, ob0)
        store(k0, ob0, semo0)

        @pl.when(k0 + 2 < nchunks)
        def _():
            prep(k0 + 2, idx0, wt0)
            fire(idx0, buf0, sem0)

        wait(idx1, buf1, sem1)

        @pl.when(kk > 0)
        def _():
            drain(k1 - 2, ob1, semo1)

        combine(wt1, buf1, ob1)
        store(k1, ob1, semo1)
        return 0

    lax.fori_loop(0, nchunks // 2, loop, 0)
    drain(nchunks - 2, ob0, semo0)
    drain(nchunks - 1, ob1, semo1)


def kernel(w, feature, view_gp_mask):
    B, H, W, C = feature.shape
    N = B * H * W
    wflat = w.reshape(N, 2)
    wx = wflat[:, 0]
    wy = wflat[:, 1]
    feat = feature.reshape(N, C)
    mask = view_gp_mask.reshape(H * W)

    mesh = plsc.VectorSubcoreMesh(core_axis_name="c", subcore_axis_name="s")
    body = functools.partial(_warp_body, B, H, W, C)
    out = pl.kernel(
        body,
        out_type=jax.ShapeDtypeStruct((N, C), jnp.float32),
        mesh=mesh,
        compiler_params=pltpu.CompilerParams(
            needs_layout_passes=False, use_tc_tiling_on_sc=False),
        scratch_types=[
            pltpu.VMEM((_P,), jnp.float32),
            pltpu.VMEM((_P,), jnp.float32),
            pltpu.VMEM((_P,), jnp.float32),
            pltpu.VMEM((4, _P), jnp.int32),
            pltpu.VMEM((4, _P), jnp.float32),
            pltpu.VMEM((4, _P, C), jnp.float32),
            pltpu.VMEM((4, _P), jnp.int32),
            pltpu.VMEM((4, _P), jnp.float32),
            pltpu.VMEM((4, _P, C), jnp.float32),
            pltpu.VMEM((_P, C), jnp.float32),
            pltpu.VMEM((_P, C), jnp.float32),
            pltpu.SemaphoreType.DMA,
            pltpu.SemaphoreType.DMA,
            pltpu.SemaphoreType.DMA,
            pltpu.SemaphoreType.DMA,
        ],
    )(wx, wy, feat, mask)
    return out.reshape(B, H, W, C)
